# Initial kernel scaffold; baseline (speedup 1.0000x reference)
import jax
import jax.numpy as jnp
from jax.experimental import pallas as pl


def kernel(x, params, edge_index):
    raise NotImplementedError("write your pallas kernel here")


# speedup vs baseline: 1.0238x; 1.0238x over previous
import functools

import jax
import jax.numpy as jnp
from jax import lax
from jax.experimental import pallas as pl
from jax.experimental.pallas import tpu as pltpu

N = 10000
E = 320000
D_IN = 128
H = 256
HEADS = 4
DH = 64
NOTES = 128
L = 3

ROW_BLK = 1000


def _ln(z):
    m = jnp.mean(z, axis=-1, keepdims=True)
    v = jnp.mean((z - m) ** 2, axis=-1, keepdims=True)
    return (z - m) * lax.rsqrt(v + 1e-5)


def _mlp_block_kernel(hn_ref, agg_ref, res_ref, eps_ref, w0_ref, b0_ref,
                      w1_ref, b1_ref, w2_ref, b2_ref, h_out_ref, hn_out_ref,
                      *, has_residual):
    z = (1.0 + eps_ref[0, 0]) * hn_ref[...] + agg_ref[...]
    z = jnp.maximum(
        jnp.dot(z, w0_ref[...], preferred_element_type=jnp.float32)
        + b0_ref[...], 0.0)
    z = jnp.maximum(
        jnp.dot(z, w1_ref[...], preferred_element_type=jnp.float32)
        + b1_ref[...], 0.0)
    z = jnp.dot(z, w2_ref[...], preferred_element_type=jnp.float32) + b2_ref[...]
    z = _ln(z)
    if has_residual:
        z = z + res_ref[...]
    h_out_ref[...] = z
    hn_out_ref[...] = _ln(z)


def _mlp_block(hn, agg, res, eps, w0, b0, w1, b1, w2, b2):
    n, in_d = hn.shape
    has_residual = res is not None
    grid = (n // ROW_BLK,)
    in_specs = [
        pl.BlockSpec((ROW_BLK, in_d), lambda i: (i, 0)),
        pl.BlockSpec((ROW_BLK, in_d), lambda i: (i, 0)),
    ]
    args = [hn, agg]
    if has_residual:
        in_specs.append(pl.BlockSpec((ROW_BLK, H), lambda i: (i, 0)))
        args.append(res)
    else:
        in_specs.append(pl.BlockSpec(memory_space=pltpu.SMEM))
        args.append(jnp.zeros((1,), jnp.float32))
    in_specs.append(pl.BlockSpec(memory_space=pltpu.SMEM))
    args.append(eps.reshape(1, 1))
    for w, b in ((w0, b0), (w1, b1), (w2, b2)):
        in_specs.append(pl.BlockSpec((w.shape[0], H), lambda i: (0, 0)))
        in_specs.append(pl.BlockSpec((1, H), lambda i: (0, 0)))
        args.extend([w, b.reshape(1, H)])
    out_specs = [
        pl.BlockSpec((ROW_BLK, H), lambda i: (i, 0)),
        pl.BlockSpec((ROW_BLK, H), lambda i: (i, 0)),
    ]
    return pl.pallas_call(
        functools.partial(_mlp_block_kernel, has_residual=has_residual),
        grid=grid,
        in_specs=in_specs,
        out_specs=out_specs,
        out_shape=[
            jax.ShapeDtypeStruct((n, H), jnp.float32),
            jax.ShapeDtypeStruct((n, H), jnp.float32),
        ],
    )(*args)


def _ln_kernel(x_ref, o_ref):
    o_ref[...] = _ln(x_ref[...])


def _ln_call(x):
    n, d = x.shape
    return pl.pallas_call(
        _ln_kernel,
        grid=(n // ROW_BLK,),
        in_specs=[pl.BlockSpec((ROW_BLK, d), lambda i: (i, 0))],
        out_specs=pl.BlockSpec((ROW_BLK, d), lambda i: (i, 0)),
        out_shape=jax.ShapeDtypeStruct((n, d), jnp.float32),
    )(x)


def kernel(x, params, edge_index):
    src = edge_index[0]
    dst = edge_index[1]

    hn = _ln_call(x)
    h = None
    residual = None
    for l in range(L):
        agg = jax.ops.segment_sum(hn[src], dst, num_segments=N)
        h, hn = _mlp_block(
            hn, agg, residual, params["eps%d" % l],
            params["W%d_0" % l], params["b%d_0" % l],
            params["W%d_1" % l], params["b%d_1" % l],
            params["W%d_2" % l], params["b%d_2" % l])
        residual = h

    k = (h @ params["Wk"]).reshape(N, HEADS, DH)
    v = (h @ params["Wv"]).reshape(N, HEADS, DH)
    scores = jnp.einsum("hd,nhd->hn", params["seed"], k) / jnp.sqrt(float(DH))
    attn = jax.nn.softmax(scores, axis=-1)
    pooled = jnp.einsum("hn,nhd->hd", attn, v).reshape(1, H)
    out = pooled @ params["Wo"]
    embed = out
    logits = out @ params["Wn"] + params["bn"]
    return (embed, logits)
